# baseline (speedup 1.0000x reference)
import functools

import jax
import jax.numpy as jnp
from jax import lax
from jax.experimental import pallas as pl
from jax.experimental.pallas import tpu as pltpu
from jax.experimental.pallas import tpu_sc as plsc

_NC = 2
_NS = 16
_NW = _NC * _NS
_CHUNK = 1000


def _prep_body(x_ref, f_ref, w1a_ref, w1b_ref, b1_ref,
               at_ref, ft_ref, b_ref):
    pb = x_ref.shape[0]
    x = x_ref[...]
    xp16 = jnp.concatenate([x, jnp.zeros((pb, 13), jnp.float32)], axis=1)
    aa = jnp.dot(xp16, w1a_ref[...], preferred_element_type=jnp.float32)
    at_ref[...] = aa.astype(jnp.bfloat16)
    ft_ref[...] = f_ref[...].astype(jnp.bfloat16)
    b_ref[...] = jnp.dot(xp16, w1b_ref[...],
                         preferred_element_type=jnp.float32) + b1_ref[...]


def _main_body(a0_ref, a1_ref, a2_ref, a3_ref,
               f0_ref, f1_ref, f2_ref, f3_ref,
               b_ref, s4_ref, w2_ref, b2_ref, o_ref):
    bt = jnp.dot(b_ref[...], s4_ref[...],
                 preferred_element_type=jnp.float32)
    w2 = w2_ref[...]
    b2 = b2_ref[...]
    acc = jnp.zeros_like(bt)
    for aq, fq in ((a0_ref, f0_ref), (a1_ref, f1_ref),
                   (a2_ref, f2_ref), (a3_ref, f3_ref)):
        h = jax.nn.gelu(aq[...].astype(jnp.float32) + bt)
        mlp = jnp.dot(h, w2, preferred_element_type=jnp.float32) + b2
        acc = acc + mlp * fq[...].astype(jnp.float32)
    r = acc[:, 0:64] + acc[:, 64:128]
    r = r[:, 0:32] + r[:, 32:64]
    o_ref[...] = r * (1.0 / 16.0)


@functools.lru_cache(maxsize=None)
def _make_sc_gather(e_total):
    epw = e_total // _NW
    nit = epw // _CHUNK
    assert epw * _NW == e_total and nit * _CHUNK == epw
    assert nit % 2 == 0 and nit >= 4
    mesh = plsc.VectorSubcoreMesh(core_axis_name="c", subcore_axis_name="s")

    @functools.partial(
        pl.kernel, mesh=mesh,
        compiler_params=pltpu.CompilerParams(use_tc_tiling_on_sc=False),
        out_type=[jax.ShapeDtypeStruct((e_total, 32), jnp.bfloat16),
                  jax.ShapeDtypeStruct((e_total, 32), jnp.bfloat16)],
        scratch_types=[pltpu.VMEM((_CHUNK,), jnp.int32),
                       pltpu.VMEM((_CHUNK, 32), jnp.bfloat16),
                       pltpu.VMEM((_CHUNK, 32), jnp.bfloat16),
                       pltpu.VMEM((_CHUNK,), jnp.int32),
                       pltpu.VMEM((_CHUNK, 32), jnp.bfloat16),
                       pltpu.VMEM((_CHUNK, 32), jnp.bfloat16),
                       pltpu.SemaphoreType.DMA,
                       pltpu.SemaphoreType.DMA,
                       pltpu.SemaphoreType.DMA,
                       pltpu.SemaphoreType.DMA],
    )
    def gather_k(atab, ftab, idx_hbm, ag_hbm, fg_hbm,
                 i0, a0, f0, i1, a1, f1, sg0, sg1, sw0, sw1):
        wid = lax.axis_index("s") * _NC + lax.axis_index("c")
        base = wid * epw
        idxs, as_, fs = (i0, i1), (a0, a1), (f0, f1)
        sgs, sws = (sg0, sg1), (sw0, sw1)

        def fire_gather(b, chunk):
            off = base + chunk * _CHUNK
            pltpu.sync_copy(idx_hbm.at[pl.ds(off, _CHUNK)], idxs[b])
            pltpu.async_copy(atab.at[idxs[b]], as_[b], sgs[b])
            pltpu.async_copy(ftab.at[idxs[b]], fs[b], sgs[b])

        def wait_gather(b):
            pltpu.make_async_copy(atab.at[idxs[b]], as_[b], sgs[b]).wait()
            pltpu.make_async_copy(ftab.at[idxs[b]], fs[b], sgs[b]).wait()

        def writeback(b, chunk):
            off = base + chunk * _CHUNK
            wa = pltpu.async_copy(as_[b], ag_hbm.at[pl.ds(off, _CHUNK)],
                                  sws[b])
            wf = pltpu.async_copy(fs[b], fg_hbm.at[pl.ds(off, _CHUNK)],
                                  sws[b])
            wa.wait()
            wf.wait()

        fire_gather(0, 0)
        fire_gather(1, 1)

        def body(it2, carry):
            for b in (0, 1):
                cur = 2 * it2 + b
                wait_gather(b)
                writeback(b, cur)
                fire_gather(b, cur + 2)
            return carry

        lax.fori_loop(0, (nit - 2) // 2, body, 0)

        for b in (0, 1):
            cur = nit - 2 + b
            wait_gather(b)
            writeback(b, cur)

    return gather_k


def kernel(x_in, in_features, W1, b1, W2, b2,
           neighbors_index, neighbors_row_splits):
    n, c = in_features.shape
    e = neighbors_index.shape[0]
    f32 = jnp.float32
    bf16 = jnp.bfloat16
    assert c == 32 and e == 16 * n and neighbors_row_splits.shape[0] == n + 1

    pb = 2000
    w1ap = jnp.zeros((16, 32), f32).at[0:3].set(W1[0:3])
    w1bp = jnp.zeros((16, 32), f32).at[0:3].set(W1[3:6])
    atab, ftab, bmat = pl.pallas_call(
        _prep_body,
        grid=(n // pb,),
        in_specs=[pl.BlockSpec((pb, 3), lambda i: (i, 0)),
                  pl.BlockSpec((pb, 32), lambda i: (i, 0)),
                  pl.BlockSpec((16, 32), lambda i: (0, 0)),
                  pl.BlockSpec((16, 32), lambda i: (0, 0)),
                  pl.BlockSpec((1, 32), lambda i: (0, 0))],
        out_specs=[pl.BlockSpec((pb, 32), lambda i: (i, 0)),
                   pl.BlockSpec((pb, 32), lambda i: (i, 0)),
                   pl.BlockSpec((pb, 32), lambda i: (i, 0))],
        out_shape=[jax.ShapeDtypeStruct((n, 32), bf16),
                   jax.ShapeDtypeStruct((n, 32), bf16),
                   jax.ShapeDtypeStruct((n, 32), f32)],
    )(x_in, in_features, w1ap, w1bp, b1.reshape(1, 32))

    idxq = neighbors_index.reshape(n, 4, 4).transpose(1, 0, 2).reshape(e)

    ag, fg = _make_sc_gather(e)(atab, ftab, idxq)

    ag4 = ag.reshape(e // 4, 128)
    fg4 = fg.reshape(e // 4, 128)
    s4 = jnp.tile(jnp.eye(32, dtype=f32), (1, 4))
    w2bd = jnp.kron(jnp.eye(4, dtype=f32), W2)
    b2t = jnp.tile(b2, 4).reshape(1, 128)

    nb = 1000
    nblk = n // nb
    qspec = [pl.BlockSpec((nb, 128), (lambda i, q=q: (q * nblk + i, 0)))
             for q in range(4)]
    out = pl.pallas_call(
        _main_body,
        grid=(nblk,),
        in_specs=(qspec + qspec
                  + [pl.BlockSpec((nb, 32), lambda i: (i, 0)),
                     pl.BlockSpec((32, 128), lambda i: (0, 0)),
                     pl.BlockSpec((128, 128), lambda i: (0, 0)),
                     pl.BlockSpec((1, 128), lambda i: (0, 0))]),
        out_specs=pl.BlockSpec((nb, 32), lambda i: (i, 0)),
        out_shape=jax.ShapeDtypeStruct((n, 32), f32),
    )(ag4, ag4, ag4, ag4, fg4, fg4, fg4, fg4, bmat, s4, w2bd, b2t)
    return out


# speedup vs baseline: 2.0083x; 2.0083x over previous
import functools

import jax
import jax.numpy as jnp
from jax import lax
from jax.experimental import pallas as pl
from jax.experimental.pallas import tpu as pltpu
from jax.experimental.pallas import tpu_sc as plsc

_NC = 2
_NS = 16
_NW = _NC * _NS
_CHUNK = 1000


def _prep_body(x_ref, f_ref, w1b_ref, b1_ref, xp_ref, lo_ref, hi_ref, b2_ref):
    pb = x_ref.shape[0]
    x = x_ref[...]
    fv = f_ref[...]
    xp = jnp.concatenate([x, jnp.zeros((pb, 13), jnp.float32)], axis=1)
    xp_ref[...] = xp
    lo_ref[...] = fv[:, 0:16]
    hi_ref[...] = fv[:, 16:32]
    bb = jnp.dot(xp, w1b_ref[...],
                 preferred_element_type=jnp.float32) + b1_ref[...]
    b2_ref[...] = jnp.broadcast_to(
        bb[:, None, :], (pb, 2, 32)).reshape(2 * pb, 32)


def _main_body(xg_ref, lo_ref, hi_ref, b2_ref, w1c_ref,
               w2lo_ref, w2hi_ref, b2lo_ref, b2hi_ref, o_ref):
    r2 = xg_ref.shape[0]
    nb = r2 // 2
    xb = jnp.concatenate([xg_ref[...], b2_ref[...]], axis=1)
    h8 = jax.nn.gelu(jnp.dot(xb, w1c_ref[...],
                             preferred_element_type=jnp.float32))
    mlo = jnp.dot(h8, w2lo_ref[...],
                  preferred_element_type=jnp.float32) + b2lo_ref[...]
    mhi = jnp.dot(h8, w2hi_ref[...],
                  preferred_element_type=jnp.float32) + b2hi_ref[...]
    wlo = mlo * lo_ref[...]
    whi = mhi * hi_ref[...]
    rlo = wlo[:, 0:64] + wlo[:, 64:128]
    rlo = rlo[:, 0:32] + rlo[:, 32:64]
    rlo = rlo[:, 0:16] + rlo[:, 16:32]
    rhi = whi[:, 0:64] + whi[:, 64:128]
    rhi = rhi[:, 0:32] + rhi[:, 32:64]
    rhi = rhi[:, 0:16] + rhi[:, 16:32]
    r = jnp.concatenate([rlo, rhi], axis=1)
    o_ref[...] = r.reshape(nb, 2, 32).sum(axis=1) * (1.0 / 16.0)


@functools.lru_cache(maxsize=None)
def _make_sc_gather(e_total):
    epw = e_total // _NW
    nit = epw // _CHUNK
    assert epw * _NW == e_total and nit * _CHUNK == epw
    assert nit % 2 == 0 and nit >= 4
    mesh = plsc.VectorSubcoreMesh(core_axis_name="c", subcore_axis_name="s")

    @functools.partial(
        pl.kernel, mesh=mesh,
        compiler_params=pltpu.CompilerParams(use_tc_tiling_on_sc=False),
        out_type=[jax.ShapeDtypeStruct((e_total, 16), jnp.float32),
                  jax.ShapeDtypeStruct((e_total, 16), jnp.float32),
                  jax.ShapeDtypeStruct((e_total, 16), jnp.float32)],
        scratch_types=[pltpu.VMEM((_CHUNK,), jnp.int32),
                       pltpu.VMEM((_CHUNK, 16), jnp.float32),
                       pltpu.VMEM((_CHUNK, 16), jnp.float32),
                       pltpu.VMEM((_CHUNK, 16), jnp.float32),
                       pltpu.VMEM((_CHUNK,), jnp.int32),
                       pltpu.VMEM((_CHUNK, 16), jnp.float32),
                       pltpu.VMEM((_CHUNK, 16), jnp.float32),
                       pltpu.VMEM((_CHUNK, 16), jnp.float32),
                       pltpu.SemaphoreType.DMA,
                       pltpu.SemaphoreType.DMA,
                       pltpu.SemaphoreType.DMA,
                       pltpu.SemaphoreType.DMA],
    )
    def gather_k(xtab, lotab, hitab, idx_hbm, xg_hbm, lo_hbm, hi_hbm,
                 i0, x0, l0, h0, i1, x1, l1, h1, sg0, sg1, sw0, sw1):
        wid = lax.axis_index("s") * _NC + lax.axis_index("c")
        base = wid * epw
        idxs, xs, ls, hs = (i0, i1), (x0, x1), (l0, l1), (h0, h1)
        sgs, sws = (sg0, sg1), (sw0, sw1)

        def fire_gather(b, chunk):
            off = base + chunk * _CHUNK
            pltpu.sync_copy(idx_hbm.at[pl.ds(off, _CHUNK)], idxs[b])
            pltpu.async_copy(xtab.at[idxs[b]], xs[b], sgs[b])
            pltpu.async_copy(lotab.at[idxs[b]], ls[b], sgs[b])
            pltpu.async_copy(hitab.at[idxs[b]], hs[b], sgs[b])

        def wait_gather(b):
            pltpu.make_async_copy(xtab.at[idxs[b]], xs[b], sgs[b]).wait()
            pltpu.make_async_copy(lotab.at[idxs[b]], ls[b], sgs[b]).wait()
            pltpu.make_async_copy(hitab.at[idxs[b]], hs[b], sgs[b]).wait()

        def writeback(b, chunk):
            off = base + chunk * _CHUNK
            ws = [pltpu.async_copy(xs[b], xg_hbm.at[pl.ds(off, _CHUNK)],
                                   sws[b]),
                  pltpu.async_copy(ls[b], lo_hbm.at[pl.ds(off, _CHUNK)],
                                   sws[b]),
                  pltpu.async_copy(hs[b], hi_hbm.at[pl.ds(off, _CHUNK)],
                                   sws[b])]
            for w in ws:
                w.wait()

        fire_gather(0, 0)
        fire_gather(1, 1)

        def body(it2, carry):
            for b in (0, 1):
                cur = 2 * it2 + b
                wait_gather(b)
                writeback(b, cur)
                fire_gather(b, cur + 2)
            return carry

        lax.fori_loop(0, (nit - 2) // 2, body, 0)

        for b in (0, 1):
            cur = nit - 2 + b
            wait_gather(b)
            writeback(b, cur)

    return gather_k


def kernel(x_in, in_features, W1, b1, W2, b2,
           neighbors_index, neighbors_row_splits):
    n, c = in_features.shape
    e = neighbors_index.shape[0]
    f32 = jnp.float32
    assert c == 32 and e == 16 * n and neighbors_row_splits.shape[0] == n + 1

    pb = 2000
    w1bp = jnp.zeros((16, 32), f32).at[0:3].set(W1[3:6])
    xpad, flo, fhi, b2d = pl.pallas_call(
        _prep_body,
        grid=(n // pb,),
        in_specs=[pl.BlockSpec((pb, 3), lambda i: (i, 0)),
                  pl.BlockSpec((pb, 32), lambda i: (i, 0)),
                  pl.BlockSpec((16, 32), lambda i: (0, 0)),
                  pl.BlockSpec((1, 32), lambda i: (0, 0))],
        out_specs=[pl.BlockSpec((pb, 16), lambda i: (i, 0)),
                   pl.BlockSpec((pb, 16), lambda i: (i, 0)),
                   pl.BlockSpec((pb, 16), lambda i: (i, 0)),
                   pl.BlockSpec((2 * pb, 32), lambda i: (i, 0))],
        out_shape=[jax.ShapeDtypeStruct((n, 16), f32),
                   jax.ShapeDtypeStruct((n, 16), f32),
                   jax.ShapeDtypeStruct((n, 16), f32),
                   jax.ShapeDtypeStruct((2 * n, 32), f32)],
    )(x_in, in_features, w1bp, b1.reshape(1, 32))

    xg, fglo, fghi = _make_sc_gather(e)(xpad, flo, fhi, neighbors_index)

    xg8 = xg.reshape(e // 8, 128)
    lo8 = fglo.reshape(e // 8, 128)
    hi8 = fghi.reshape(e // 8, 128)

    eye8 = jnp.eye(8, dtype=f32)
    w1blk = jnp.zeros((16, 32), f32).at[0:3].set(W1[0:3])
    w1cat = jnp.concatenate(
        [jnp.kron(eye8, w1blk),
         jnp.tile(jnp.eye(32, dtype=f32), (1, 8))], axis=0)
    w2lo = jnp.kron(eye8, W2[:, 0:16])
    w2hi = jnp.kron(eye8, W2[:, 16:32])
    b2lo = jnp.tile(b2[0:16], 8).reshape(1, 128)
    b2hi = jnp.tile(b2[16:32], 8).reshape(1, 128)

    nb = 1000
    out = pl.pallas_call(
        _main_body,
        grid=(n // nb,),
        in_specs=[pl.BlockSpec((2 * nb, 128), lambda i: (i, 0)),
                  pl.BlockSpec((2 * nb, 128), lambda i: (i, 0)),
                  pl.BlockSpec((2 * nb, 128), lambda i: (i, 0)),
                  pl.BlockSpec((2 * nb, 32), lambda i: (i, 0)),
                  pl.BlockSpec((160, 256), lambda i: (0, 0)),
                  pl.BlockSpec((256, 128), lambda i: (0, 0)),
                  pl.BlockSpec((256, 128), lambda i: (0, 0)),
                  pl.BlockSpec((1, 128), lambda i: (0, 0)),
                  pl.BlockSpec((1, 128), lambda i: (0, 0))],
        out_specs=pl.BlockSpec((nb, 32), lambda i: (i, 0)),
        out_shape=jax.ShapeDtypeStruct((n, 32), f32),
    )(xg8, lo8, hi8, b2d, w1cat, w2lo, w2hi, b2lo, b2hi)
    return out
